# Initial kernel scaffold; baseline (speedup 1.0000x reference)
import jax
import jax.numpy as jnp
from jax.experimental import pallas as pl


def kernel(x, edge_index, edge_attr, We, be, W1, b1, W2, b2):
    raise NotImplementedError("write your pallas kernel here")


# speedup vs baseline: 3.3456x; 3.3456x over previous
import functools

import jax
import jax.numpy as jnp
from jax import lax
from jax.experimental import pallas as pl
from jax.experimental.pallas import tpu as pltpu
from jax.experimental.pallas import tpu_sc as plsc

N = 10000
E = 320000
D = 128
NC = 2
NS = 16
C = 80
RPT = 640
N_PAD = RPT * NS
EA = E // (NC * NS)
EB = E // NS

_mesh = plsc.VectorSubcoreMesh(
    core_axis_name="c", subcore_axis_name="s", num_cores=NC, num_subcores=NS
)


@functools.partial(
    pl.kernel,
    out_type=(
        jax.ShapeDtypeStruct((NC, N_PAD, D), jnp.float32),
        jax.ShapeDtypeStruct((NC * NS, N_PAD), jnp.float32),
    ),
    mesh=_mesh,
    compiler_params=pltpu.CompilerParams(needs_layout_passes=False),
    scratch_types=(
        pltpu.VMEM_SHARED((N_PAD, D), jnp.float32),
        pltpu.VMEM((C,), jnp.int32),
        pltpu.VMEM((C, D), jnp.float32),
        pltpu.VMEM((N_PAD,), jnp.float32),
    ),
)
def _sc_edge_agg(ea_hbm, dst_hbm, z128_hbm, zdeg_hbm,
                 psum_hbm, pdeg_hbm,
                 acc, idx_v, rows_v, deg_t):
    c = lax.axis_index("c")
    s = lax.axis_index("s")
    pltpu.sync_copy(z128_hbm, rows_v)
    for j in range(RPT // C):
        pltpu.sync_copy(rows_v, acc.at[pl.ds(s * RPT + j * C, C)])
    pltpu.sync_copy(zdeg_hbm, deg_t)
    plsc.subcore_barrier()

    base0 = (c * NS + s) * EA
    ones_vec = jnp.ones((16,), jnp.float32)

    def chunk(k, _):
        base = base0 + k * C
        pltpu.sync_copy(dst_hbm.at[pl.ds(base, C)], idx_v)
        pltpu.sync_copy(ea_hbm.at[pl.ds(base, C)], rows_v)
        pltpu.sync_copy(rows_v, acc.at[idx_v], add=True)
        for j in range(C // 16):
            plsc.addupdate_scatter(deg_t, [idx_v[pl.ds(16 * j, 16)]], ones_vec)
        return _

    lax.fori_loop(0, EA // C, chunk, None)
    plsc.subcore_barrier()

    for j in range(RPT // C):
        r = s * RPT + j * C
        pltpu.sync_copy(acc.at[pl.ds(r, C)], rows_v)
        pltpu.sync_copy(rows_v, psum_hbm.at[c, pl.ds(r, C)])
    pltpu.sync_copy(deg_t, pdeg_hbm.at[c * NS + s])


@functools.partial(
    pl.kernel,
    out_type=jax.ShapeDtypeStruct((NC, N_PAD, D), jnp.float32),
    mesh=_mesh,
    scratch_types=(
        pltpu.VMEM_SHARED((N_PAD, D), jnp.float32),
        pltpu.VMEM((C,), jnp.int32),
        pltpu.VMEM((C,), jnp.int32),
        pltpu.VMEM((C, D), jnp.float32),
    ),
)
def _sc_neigh(x_hbm, agg_hbm, src_hbm, dst_hbm, z128_hbm,
              neigh_hbm,
              acc, idxs_v, idxd_v, rows_v):
    c = lax.axis_index("c")
    s = lax.axis_index("s")
    pltpu.sync_copy(z128_hbm, rows_v)
    for j in range(RPT // C):
        pltpu.sync_copy(rows_v, acc.at[pl.ds(s * RPT + j * C, C)])
    plsc.subcore_barrier()

    base0 = s * EB

    def make_loop(table_hbm):
        def chunk(k, _):
            base = base0 + k * C
            pltpu.sync_copy(src_hbm.at[pl.ds(base, C)], idxs_v)
            pltpu.sync_copy(dst_hbm.at[pl.ds(base, C)], idxd_v)
            pltpu.sync_copy(table_hbm.at[idxs_v], rows_v)
            pltpu.sync_copy(rows_v, acc.at[idxd_v], add=True)
            return _
        return chunk

    @pl.when(c == 0)
    def _():
        lax.fori_loop(0, EB // C, make_loop(x_hbm), None)

    @pl.when(c == 1)
    def _():
        lax.fori_loop(0, EB // C, make_loop(agg_hbm), None)

    plsc.subcore_barrier()
    for j in range(RPT // C):
        r = s * RPT + j * C
        pltpu.sync_copy(acc.at[pl.ds(r, C)], rows_v)
        pltpu.sync_copy(rows_v, neigh_hbm.at[c, pl.ds(r, C)])


_RB = 1000


def _tc_combine_body(psum_ref, pdeg_ref, we_ref, be_ref, out_ref):
    ssum = psum_ref[0] + psum_ref[1]
    deg = jnp.sum(pdeg_ref[...], axis=0)[:, None]
    out_ref[...] = (
        jnp.dot(ssum, we_ref[...], preferred_element_type=jnp.float32)
        + deg * be_ref[...]
    )


def _tc_mlp_body(x_ref, agg_ref, n0_ref, n1_ref, w1a_ref, w1b_ref, b1_ref,
                 w2_ref, b2_ref, out_ref):
    pre0 = x_ref[...] + n0_ref[...]
    pre1 = agg_ref[...] + n1_ref[...]
    h = jax.nn.relu(
        jnp.dot(pre0, w1a_ref[...], preferred_element_type=jnp.float32)
        + jnp.dot(pre1, w1b_ref[...], preferred_element_type=jnp.float32)
        + b1_ref[...]
    )
    out_ref[...] = (
        jnp.dot(h, w2_ref[...], preferred_element_type=jnp.float32)
        + b2_ref[...]
    )


def kernel(x, edge_index, edge_attr, We, be, W1, b1, W2, b2):
    src = edge_index[0]
    dst = edge_index[1]
    z128 = jnp.zeros((C, D), jnp.float32)
    zdeg = jnp.zeros((N_PAD,), jnp.float32)

    psum, pdeg = _sc_edge_agg(edge_attr, dst, z128, zdeg)

    _CB = 1024
    agg = pl.pallas_call(
        _tc_combine_body,
        grid=(N_PAD // _CB,),
        in_specs=[
            pl.BlockSpec((NC, _CB, D), lambda i: (0, i, 0)),
            pl.BlockSpec((NC * NS, _CB), lambda i: (0, i)),
            pl.BlockSpec((D, D), lambda i: (0, 0)),
            pl.BlockSpec((1, D), lambda i: (0, 0)),
        ],
        out_specs=pl.BlockSpec((_CB, D), lambda i: (i, 0)),
        out_shape=jax.ShapeDtypeStruct((N_PAD, D), jnp.float32),
    )(psum, pdeg, We, be.reshape(1, D))

    neigh = _sc_neigh(x, agg, src, dst, z128)

    out = pl.pallas_call(
        _tc_mlp_body,
        grid=(N // _RB,),
        in_specs=[
            pl.BlockSpec((_RB, D), lambda i: (i, 0)),
            pl.BlockSpec((_RB, D), lambda i: (i, 0)),
            pl.BlockSpec((_RB, D), lambda i: (i, 0)),
            pl.BlockSpec((_RB, D), lambda i: (i, 0)),
            pl.BlockSpec((D, D), lambda i: (0, 0)),
            pl.BlockSpec((D, D), lambda i: (0, 0)),
            pl.BlockSpec((1, D), lambda i: (0, 0)),
            pl.BlockSpec((D, D), lambda i: (0, 0)),
            pl.BlockSpec((1, D), lambda i: (0, 0)),
        ],
        out_specs=pl.BlockSpec((_RB, D), lambda i: (i, 0)),
        out_shape=jax.ShapeDtypeStruct((N, D), jnp.float32),
    )(x, agg, neigh[0], neigh[1], W1[:D], W1[D:], b1.reshape(1, D),
      W2, b2.reshape(1, D))

    return out


# speedup vs baseline: 6.0079x; 1.7957x over previous
import functools

import jax
import jax.numpy as jnp
from jax import lax
from jax.experimental import pallas as pl
from jax.experimental.pallas import tpu as pltpu
from jax.experimental.pallas import tpu_sc as plsc

N = 10000
E = 320000
D = 128
NC = 2
NS = 16
C = 80
NB_A = 3
NB_B = 4
RPT = 640
N_PAD = RPT * NS
EA = E // (NC * NS)
EB = E // NS

_mesh = plsc.VectorSubcoreMesh(
    core_axis_name="c", subcore_axis_name="s", num_cores=NC, num_subcores=NS
)


@functools.partial(
    pl.kernel,
    out_type=(
        jax.ShapeDtypeStruct((NC, N_PAD, D), jnp.float32),
        jax.ShapeDtypeStruct((NC * NS, N_PAD), jnp.float32),
    ),
    mesh=_mesh,
    compiler_params=pltpu.CompilerParams(needs_layout_passes=False),
    scratch_types=(
        pltpu.VMEM_SHARED((N_PAD, D), jnp.float32),
        pltpu.VMEM((NB_A, C), jnp.int32),
        pltpu.VMEM((NB_A, C, D), jnp.float32),
        pltpu.VMEM((N_PAD,), jnp.float32),
        pltpu.SemaphoreType.DMA,
        pltpu.SemaphoreType.DMA,
    ),
)
def _sc_edge_agg(ea_hbm, dst_hbm, z128_hbm, zdeg_hbm,
                 psum_hbm, pdeg_hbm,
                 acc, idx_v, rows_v, deg_t, sem_ld, sem_sc):
    c = lax.axis_index("c")
    s = lax.axis_index("s")
    pltpu.sync_copy(z128_hbm, rows_v.at[0])
    for j in range(RPT // C):
        pltpu.sync_copy(rows_v.at[0], acc.at[pl.ds(s * RPT + j * C, C)])
    pltpu.sync_copy(zdeg_hbm, deg_t)
    plsc.subcore_barrier()

    base0 = (c * NS + s) * EA
    ones_vec = jnp.ones((16,), jnp.float32)

    def do_deg(b):
        for j in range(C // 16):
            plsc.addupdate_scatter(
                deg_t, [idx_v[b, pl.ds(16 * j, 16)]], ones_vec)

    def group(g, _):
        ds_ = []
        for b in range(NB_A):
            base = base0 + (g * NB_A + b) * C
            ds_.append(pltpu.async_copy(
                dst_hbm.at[pl.ds(base, C)], idx_v.at[b], sem_ld))
            ds_.append(pltpu.async_copy(
                ea_hbm.at[pl.ds(base, C)], rows_v.at[b], sem_ld))
        for d in ds_:
            d.wait()
        ds_ = []
        for b in range(NB_A):
            ds_.append(pltpu.async_copy(
                rows_v.at[b], acc.at[idx_v.at[b]], sem_sc, add=True))
            do_deg(b)
        for d in ds_:
            d.wait()
        return _

    lax.fori_loop(0, EA // C // NB_A, group, None)
    for t in range((EA // C // NB_A) * NB_A, EA // C):
        base = base0 + t * C
        pltpu.sync_copy(dst_hbm.at[pl.ds(base, C)], idx_v.at[0])
        pltpu.sync_copy(ea_hbm.at[pl.ds(base, C)], rows_v.at[0])
        pltpu.sync_copy(rows_v.at[0], acc.at[idx_v.at[0]], add=True)
        do_deg(0)
    plsc.subcore_barrier()

    ds_ = []
    for j in range(RPT // C):
        r = s * RPT + j * C
        b = j % NB_A
        pltpu.sync_copy(acc.at[pl.ds(r, C)], rows_v.at[b])
        ds_.append(pltpu.async_copy(
            rows_v.at[b], psum_hbm.at[c, pl.ds(r, C)], sem_sc))
        if b == NB_A - 1:
            for d in ds_:
                d.wait()
            ds_ = []
    for d in ds_:
        d.wait()
    pltpu.sync_copy(deg_t, pdeg_hbm.at[c * NS + s])


@functools.partial(
    pl.kernel,
    out_type=jax.ShapeDtypeStruct((NC, N_PAD, D), jnp.float32),
    mesh=_mesh,
    scratch_types=(
        pltpu.VMEM_SHARED((N_PAD, D), jnp.float32),
        pltpu.VMEM((NB_B, C), jnp.int32),
        pltpu.VMEM((NB_B, C), jnp.int32),
        pltpu.VMEM((NB_B, C, D), jnp.float32),
        pltpu.SemaphoreType.DMA,
        pltpu.SemaphoreType.DMA,
        pltpu.SemaphoreType.DMA,
    ),
)
def _sc_neigh(x_hbm, agg_hbm, src_hbm, dst_hbm, z128_hbm,
              neigh_hbm,
              acc, idxs_v, idxd_v, rows_v, sem_ld, sem_g, sem_sc):
    c = lax.axis_index("c")
    s = lax.axis_index("s")
    pltpu.sync_copy(z128_hbm, rows_v.at[0])
    for j in range(RPT // C):
        pltpu.sync_copy(rows_v.at[0], acc.at[pl.ds(s * RPT + j * C, C)])
    plsc.subcore_barrier()

    base0 = s * EB

    def make_group(table_hbm):
        def group(g, _):
            ds_ = []
            for b in range(NB_B):
                base = base0 + (g * NB_B + b) * C
                ds_.append(pltpu.async_copy(
                    src_hbm.at[pl.ds(base, C)], idxs_v.at[b], sem_ld))
                ds_.append(pltpu.async_copy(
                    dst_hbm.at[pl.ds(base, C)], idxd_v.at[b], sem_ld))
            for d in ds_:
                d.wait()
            ds_ = []
            for b in range(NB_B):
                ds_.append(pltpu.async_copy(
                    table_hbm.at[idxs_v.at[b]], rows_v.at[b], sem_g))
            for d in ds_:
                d.wait()
            ds_ = []
            for b in range(NB_B):
                ds_.append(pltpu.async_copy(
                    rows_v.at[b], acc.at[idxd_v.at[b]], sem_sc, add=True))
            for d in ds_:
                d.wait()
            return _
        return group

    def tail(table_hbm):
        for t in range((EB // C // NB_B) * NB_B, EB // C):
            base = base0 + t * C
            pltpu.sync_copy(src_hbm.at[pl.ds(base, C)], idxs_v.at[0])
            pltpu.sync_copy(dst_hbm.at[pl.ds(base, C)], idxd_v.at[0])
            pltpu.sync_copy(table_hbm.at[idxs_v.at[0]], rows_v.at[0])
            pltpu.sync_copy(rows_v.at[0], acc.at[idxd_v.at[0]], add=True)

    @pl.when(c == 0)
    def _():
        lax.fori_loop(0, EB // C // NB_B, make_group(x_hbm), None)
        tail(x_hbm)

    @pl.when(c == 1)
    def _():
        lax.fori_loop(0, EB // C // NB_B, make_group(agg_hbm), None)
        tail(agg_hbm)

    plsc.subcore_barrier()
    ds_ = []
    for j in range(RPT // C):
        r = s * RPT + j * C
        b = j % NB_B
        pltpu.sync_copy(acc.at[pl.ds(r, C)], rows_v.at[b])
        ds_.append(pltpu.async_copy(
            rows_v.at[b], neigh_hbm.at[c, pl.ds(r, C)], sem_sc))
        if b == NB_B - 1:
            for d in ds_:
                d.wait()
            ds_ = []
    for d in ds_:
        d.wait()


_RB = 1000


def _tc_combine_body(psum_ref, pdeg_ref, we_ref, be_ref, out_ref):
    ssum = psum_ref[0] + psum_ref[1]
    deg = jnp.sum(pdeg_ref[...], axis=0)[:, None]
    out_ref[...] = (
        jnp.dot(ssum, we_ref[...], preferred_element_type=jnp.float32)
        + deg * be_ref[...]
    )


def _tc_mlp_body(x_ref, agg_ref, n0_ref, n1_ref, w1a_ref, w1b_ref, b1_ref,
                 w2_ref, b2_ref, out_ref):
    pre0 = x_ref[...] + n0_ref[...]
    pre1 = agg_ref[...] + n1_ref[...]
    h = jax.nn.relu(
        jnp.dot(pre0, w1a_ref[...], preferred_element_type=jnp.float32)
        + jnp.dot(pre1, w1b_ref[...], preferred_element_type=jnp.float32)
        + b1_ref[...]
    )
    out_ref[...] = (
        jnp.dot(h, w2_ref[...], preferred_element_type=jnp.float32)
        + b2_ref[...]
    )


def kernel(x, edge_index, edge_attr, We, be, W1, b1, W2, b2):
    src = edge_index[0]
    dst = edge_index[1]
    z128 = jnp.zeros((C, D), jnp.float32)
    zdeg = jnp.zeros((N_PAD,), jnp.float32)

    psum, pdeg = _sc_edge_agg(edge_attr, dst, z128, zdeg)

    _CB = 1024
    agg = pl.pallas_call(
        _tc_combine_body,
        grid=(N_PAD // _CB,),
        in_specs=[
            pl.BlockSpec((NC, _CB, D), lambda i: (0, i, 0)),
            pl.BlockSpec((NC * NS, _CB), lambda i: (0, i)),
            pl.BlockSpec((D, D), lambda i: (0, 0)),
            pl.BlockSpec((1, D), lambda i: (0, 0)),
        ],
        out_specs=pl.BlockSpec((_CB, D), lambda i: (i, 0)),
        out_shape=jax.ShapeDtypeStruct((N_PAD, D), jnp.float32),
    )(psum, pdeg, We, be.reshape(1, D))

    neigh = _sc_neigh(x, agg, src, dst, z128)

    out = pl.pallas_call(
        _tc_mlp_body,
        grid=(N // _RB,),
        in_specs=[
            pl.BlockSpec((_RB, D), lambda i: (i, 0)),
            pl.BlockSpec((_RB, D), lambda i: (i, 0)),
            pl.BlockSpec((_RB, D), lambda i: (i, 0)),
            pl.BlockSpec((_RB, D), lambda i: (i, 0)),
            pl.BlockSpec((D, D), lambda i: (0, 0)),
            pl.BlockSpec((D, D), lambda i: (0, 0)),
            pl.BlockSpec((1, D), lambda i: (0, 0)),
            pl.BlockSpec((D, D), lambda i: (0, 0)),
            pl.BlockSpec((1, D), lambda i: (0, 0)),
        ],
        out_specs=pl.BlockSpec((_RB, D), lambda i: (i, 0)),
        out_shape=jax.ShapeDtypeStruct((N, D), jnp.float32),
    )(x, agg, neigh[0], neigh[1], W1[:D], W1[D:], b1.reshape(1, D),
      W2, b2.reshape(1, D))

    return out


# speedup vs baseline: 7.0884x; 1.1798x over previous
import functools

import jax
import jax.numpy as jnp
from jax import lax
from jax.experimental import pallas as pl
from jax.experimental.pallas import tpu as pltpu
from jax.experimental.pallas import tpu_sc as plsc

N = 10000
E = 320000
D = 128
NC = 2
NS = 16
C = 80
NB_A = 3
NB_B = 4
RPT = 640
N_PAD = RPT * NS
EA = E // (NC * NS)
EB = E // NS

_mesh = plsc.VectorSubcoreMesh(
    core_axis_name="c", subcore_axis_name="s", num_cores=NC, num_subcores=NS
)


@functools.partial(
    pl.kernel,
    out_type=(
        jax.ShapeDtypeStruct((NC, N_PAD, D), jnp.float32),
        jax.ShapeDtypeStruct((NC * NS, N_PAD), jnp.float32),
    ),
    mesh=_mesh,
    compiler_params=pltpu.CompilerParams(needs_layout_passes=False),
    scratch_types=(
        pltpu.VMEM_SHARED((N_PAD, D), jnp.float32),
        pltpu.VMEM((NB_A, C), jnp.int32),
        pltpu.VMEM((NB_A, C, D), jnp.float32),
        pltpu.VMEM((N_PAD,), jnp.float32),
        pltpu.SemaphoreType.DMA,
        pltpu.SemaphoreType.DMA,
    ),
)
def _sc_edge_agg(ea_hbm, dst_hbm, z128_hbm, zdeg_hbm,
                 psum_hbm, pdeg_hbm,
                 acc, idx_v, rows_v, deg_t, sem_ld, sem_sc):
    c = lax.axis_index("c")
    s = lax.axis_index("s")
    pltpu.sync_copy(z128_hbm, rows_v.at[0])
    for j in range(RPT // C):
        pltpu.sync_copy(rows_v.at[0], acc.at[pl.ds(s * RPT + j * C, C)])
    pltpu.sync_copy(zdeg_hbm, deg_t)
    plsc.subcore_barrier()

    base0 = (c * NS + s) * EA
    ones_vec = jnp.ones((16,), jnp.float32)

    def do_deg(b):
        for j in range(C // 16):
            plsc.addupdate_scatter(
                deg_t, [idx_v[b, pl.ds(16 * j, 16)]], ones_vec)

    def group(g, _):
        lds = []
        for b in range(NB_A):
            base = base0 + (g * NB_A + b) * C
            lds.append((pltpu.async_copy(
                dst_hbm.at[pl.ds(base, C)], idx_v.at[b], sem_ld),
                pltpu.async_copy(
                ea_hbm.at[pl.ds(base, C)], rows_v.at[b], sem_ld)))
        scs = []
        for b in range(NB_A):
            lds[b][0].wait()
            lds[b][1].wait()
            scs.append(pltpu.async_copy(
                rows_v.at[b], acc.at[idx_v.at[b]], sem_sc, add=True))
            do_deg(b)
        for d in scs:
            d.wait()
        return _

    lax.fori_loop(0, EA // C // NB_A, group, None)
    for t in range((EA // C // NB_A) * NB_A, EA // C):
        base = base0 + t * C
        pltpu.sync_copy(dst_hbm.at[pl.ds(base, C)], idx_v.at[0])
        pltpu.sync_copy(ea_hbm.at[pl.ds(base, C)], rows_v.at[0])
        pltpu.sync_copy(rows_v.at[0], acc.at[idx_v.at[0]], add=True)
        do_deg(0)
    plsc.subcore_barrier()

    ds_ = []
    for j in range(RPT // C):
        r = s * RPT + j * C
        b = j % NB_A
        pltpu.sync_copy(acc.at[pl.ds(r, C)], rows_v.at[b])
        ds_.append(pltpu.async_copy(
            rows_v.at[b], psum_hbm.at[c, pl.ds(r, C)], sem_sc))
        if b == NB_A - 1:
            for d in ds_:
                d.wait()
            ds_ = []
    for d in ds_:
        d.wait()
    pltpu.sync_copy(deg_t, pdeg_hbm.at[c * NS + s])


@functools.partial(
    pl.kernel,
    out_type=jax.ShapeDtypeStruct((NC, N_PAD, D), jnp.float32),
    mesh=_mesh,
    scratch_types=(
        pltpu.VMEM_SHARED((N_PAD, D), jnp.float32),
        pltpu.VMEM((NB_B, C), jnp.int32),
        pltpu.VMEM((NB_B, C), jnp.int32),
        pltpu.VMEM((NB_B, C, D), jnp.float32),
        pltpu.SemaphoreType.DMA,
        pltpu.SemaphoreType.DMA,
        pltpu.SemaphoreType.DMA,
    ),
)
def _sc_neigh(x_hbm, agg_hbm, src_hbm, dst_hbm, z128_hbm,
              neigh_hbm,
              acc, idxs_v, idxd_v, rows_v, sem_ld, sem_g, sem_sc):
    c = lax.axis_index("c")
    s = lax.axis_index("s")
    pltpu.sync_copy(z128_hbm, rows_v.at[0])
    for j in range(RPT // C):
        pltpu.sync_copy(rows_v.at[0], acc.at[pl.ds(s * RPT + j * C, C)])
    plsc.subcore_barrier()

    base0 = s * EB

    def make_group(table_hbm):
        def group(g, _):
            lds = []
            for b in range(NB_B):
                base = base0 + (g * NB_B + b) * C
                lds.append((pltpu.async_copy(
                    src_hbm.at[pl.ds(base, C)], idxs_v.at[b], sem_ld),
                    pltpu.async_copy(
                    dst_hbm.at[pl.ds(base, C)], idxd_v.at[b], sem_ld)))
            gds = []
            for b in range(NB_B):
                lds[b][0].wait()
                gds.append(pltpu.async_copy(
                    table_hbm.at[idxs_v.at[b]], rows_v.at[b], sem_g))
            scs = []
            for b in range(NB_B):
                gds[b].wait()
                lds[b][1].wait()
                scs.append(pltpu.async_copy(
                    rows_v.at[b], acc.at[idxd_v.at[b]], sem_sc, add=True))
            for d in scs:
                d.wait()
            return _
        return group

    def tail(table_hbm):
        for t in range((EB // C // NB_B) * NB_B, EB // C):
            base = base0 + t * C
            pltpu.sync_copy(src_hbm.at[pl.ds(base, C)], idxs_v.at[0])
            pltpu.sync_copy(dst_hbm.at[pl.ds(base, C)], idxd_v.at[0])
            pltpu.sync_copy(table_hbm.at[idxs_v.at[0]], rows_v.at[0])
            pltpu.sync_copy(rows_v.at[0], acc.at[idxd_v.at[0]], add=True)

    @pl.when(c == 0)
    def _():
        lax.fori_loop(0, EB // C // NB_B, make_group(x_hbm), None)
        tail(x_hbm)

    @pl.when(c == 1)
    def _():
        lax.fori_loop(0, EB // C // NB_B, make_group(agg_hbm), None)
        tail(agg_hbm)

    plsc.subcore_barrier()
    ds_ = []
    for j in range(RPT // C):
        r = s * RPT + j * C
        b = j % NB_B
        pltpu.sync_copy(acc.at[pl.ds(r, C)], rows_v.at[b])
        ds_.append(pltpu.async_copy(
            rows_v.at[b], neigh_hbm.at[c, pl.ds(r, C)], sem_sc))
        if b == NB_B - 1:
            for d in ds_:
                d.wait()
            ds_ = []
    for d in ds_:
        d.wait()


_RB = 1000


def _tc_combine_body(psum_ref, pdeg_ref, we_ref, be_ref, out_ref):
    ssum = psum_ref[0] + psum_ref[1]
    deg = jnp.sum(pdeg_ref[...], axis=0)[:, None]
    out_ref[...] = (
        jnp.dot(ssum, we_ref[...], preferred_element_type=jnp.float32)
        + deg * be_ref[...]
    )


def _tc_mlp_body(x_ref, agg_ref, n0_ref, n1_ref, w1a_ref, w1b_ref, b1_ref,
                 w2_ref, b2_ref, out_ref):
    pre0 = x_ref[...] + n0_ref[...]
    pre1 = agg_ref[...] + n1_ref[...]
    h = jax.nn.relu(
        jnp.dot(pre0, w1a_ref[...], preferred_element_type=jnp.float32)
        + jnp.dot(pre1, w1b_ref[...], preferred_element_type=jnp.float32)
        + b1_ref[...]
    )
    out_ref[...] = (
        jnp.dot(h, w2_ref[...], preferred_element_type=jnp.float32)
        + b2_ref[...]
    )


def kernel(x, edge_index, edge_attr, We, be, W1, b1, W2, b2):
    src = edge_index[0]
    dst = edge_index[1]
    z128 = jnp.zeros((C, D), jnp.float32)
    zdeg = jnp.zeros((N_PAD,), jnp.float32)

    psum, pdeg = _sc_edge_agg(edge_attr, dst, z128, zdeg)

    _CB = 1024
    agg = pl.pallas_call(
        _tc_combine_body,
        grid=(N_PAD // _CB,),
        in_specs=[
            pl.BlockSpec((NC, _CB, D), lambda i: (0, i, 0)),
            pl.BlockSpec((NC * NS, _CB), lambda i: (0, i)),
            pl.BlockSpec((D, D), lambda i: (0, 0)),
            pl.BlockSpec((1, D), lambda i: (0, 0)),
        ],
        out_specs=pl.BlockSpec((_CB, D), lambda i: (i, 0)),
        out_shape=jax.ShapeDtypeStruct((N_PAD, D), jnp.float32),
    )(psum, pdeg, We, be.reshape(1, D))

    neigh = _sc_neigh(x, agg, src, dst, z128)

    out = pl.pallas_call(
        _tc_mlp_body,
        grid=(N // _RB,),
        in_specs=[
            pl.BlockSpec((_RB, D), lambda i: (i, 0)),
            pl.BlockSpec((_RB, D), lambda i: (i, 0)),
            pl.BlockSpec((_RB, D), lambda i: (i, 0)),
            pl.BlockSpec((_RB, D), lambda i: (i, 0)),
            pl.BlockSpec((D, D), lambda i: (0, 0)),
            pl.BlockSpec((D, D), lambda i: (0, 0)),
            pl.BlockSpec((1, D), lambda i: (0, 0)),
            pl.BlockSpec((D, D), lambda i: (0, 0)),
            pl.BlockSpec((1, D), lambda i: (0, 0)),
        ],
        out_specs=pl.BlockSpec((_RB, D), lambda i: (i, 0)),
        out_shape=jax.ShapeDtypeStruct((N, D), jnp.float32),
    )(x, agg, neigh[0], neigh[1], W1[:D], W1[D:], b1.reshape(1, D),
      W2, b2.reshape(1, D))

    return out


# speedup vs baseline: 7.1492x; 1.0086x over previous
import functools

import jax
import jax.numpy as jnp
from jax import lax
from jax.experimental import pallas as pl
from jax.experimental.pallas import tpu as pltpu
from jax.experimental.pallas import tpu_sc as plsc

N = 10000
E = 320000
D = 128
NC = 2
NS = 16
C = 80
NB_A = 3
NB_B = 4
CPP_A = 25
CPP_B = 25
RPT = 640
N_PAD = RPT * NS
EA = E // (NC * NS)
EB = E // NS

_mesh = plsc.VectorSubcoreMesh(
    core_axis_name="c", subcore_axis_name="s", num_cores=NC, num_subcores=NS
)


@functools.partial(
    pl.kernel,
    out_type=(
        jax.ShapeDtypeStruct((NC, N_PAD, D), jnp.float32),
        jax.ShapeDtypeStruct((NC * NS, N_PAD), jnp.float32),
    ),
    mesh=_mesh,
    compiler_params=pltpu.CompilerParams(needs_layout_passes=False),
    scratch_types=(
        pltpu.VMEM_SHARED((N_PAD, D), jnp.float32),
        pltpu.VMEM((CPP_A, C), jnp.int32),
        pltpu.VMEM((NB_A, C, D), jnp.float32),
        pltpu.VMEM((N_PAD,), jnp.float32),
        pltpu.SemaphoreType.DMA,
        pltpu.SemaphoreType.DMA,
    ),
)
def _sc_edge_agg(ea_hbm, dst3_hbm, z128_hbm, zdeg_hbm,
                 psum_hbm, pdeg_hbm,
                 acc, idx_v, rows_v, deg_t, sem_ld, sem_sc):
    c = lax.axis_index("c")
    s = lax.axis_index("s")
    pltpu.sync_copy(z128_hbm, rows_v.at[0])
    for j in range(RPT // C):
        pltpu.sync_copy(rows_v.at[0], acc.at[pl.ds(s * RPT + j * C, C)])
    pltpu.sync_copy(zdeg_hbm, deg_t)
    plsc.subcore_barrier()

    blk0 = (c * NS + s) * ((EA // C) // CPP_A)
    ones_vec = jnp.ones((16,), jnp.float32)

    def do_deg(j):
        for v in range(C // 16):
            plsc.addupdate_scatter(
                deg_t, [idx_v[j, pl.ds(16 * v, 16)]], ones_vec)

    def phase(ph, _):
        pltpu.sync_copy(dst3_hbm.at[blk0 + ph], idx_v)

        def group(g, _):
            lds = []
            for b in range(NB_A):
                ck = (blk0 + ph) * CPP_A + g * NB_A + b
                lds.append(pltpu.async_copy(
                    ea_hbm.at[pl.ds(ck * C, C)], rows_v.at[b], sem_ld))
            scs = []
            for b in range(NB_A):
                j = g * NB_A + b
                lds[b].wait()
                scs.append(pltpu.async_copy(
                    rows_v.at[b], acc.at[idx_v.at[j]], sem_sc, add=True))
                do_deg(j)
            for d in scs:
                d.wait()
            return _

        lax.fori_loop(0, CPP_A // NB_A, group, None)
        for t in range((CPP_A // NB_A) * NB_A, CPP_A):
            ck = (blk0 + ph) * CPP_A + t
            pltpu.sync_copy(ea_hbm.at[pl.ds(ck * C, C)], rows_v.at[0])
            pltpu.sync_copy(rows_v.at[0], acc.at[idx_v.at[t]], add=True)
            do_deg(t)
        return _

    lax.fori_loop(0, (EA // C) // CPP_A, phase, None)
    plsc.subcore_barrier()

    ds_ = []
    for j in range(RPT // C):
        r = s * RPT + j * C
        b = j % NB_A
        pltpu.sync_copy(acc.at[pl.ds(r, C)], rows_v.at[b])
        ds_.append(pltpu.async_copy(
            rows_v.at[b], psum_hbm.at[c, pl.ds(r, C)], sem_sc))
        if b == NB_A - 1:
            for d in ds_:
                d.wait()
            ds_ = []
    for d in ds_:
        d.wait()
    pltpu.sync_copy(deg_t, pdeg_hbm.at[c * NS + s])


@functools.partial(
    pl.kernel,
    out_type=jax.ShapeDtypeStruct((NC, N_PAD, D), jnp.float32),
    mesh=_mesh,
    scratch_types=(
        pltpu.VMEM_SHARED((N_PAD, D), jnp.float32),
        pltpu.VMEM((CPP_B, C), jnp.int32),
        pltpu.VMEM((CPP_B, C), jnp.int32),
        pltpu.VMEM((NB_B, C, D), jnp.float32),
        pltpu.SemaphoreType.DMA,
        pltpu.SemaphoreType.DMA,
    ),
)
def _sc_neigh(x_hbm, agg_hbm, src3_hbm, dst3_hbm, z128_hbm,
              neigh_hbm,
              acc, idxs_v, idxd_v, rows_v, sem_g, sem_sc):
    c = lax.axis_index("c")
    s = lax.axis_index("s")
    pltpu.sync_copy(z128_hbm, rows_v.at[0])
    for j in range(RPT // C):
        pltpu.sync_copy(rows_v.at[0], acc.at[pl.ds(s * RPT + j * C, C)])
    plsc.subcore_barrier()

    blk0 = s * ((EB // C) // CPP_B)

    def make_phase(table_hbm):
        def phase(ph, _):
            pltpu.sync_copy(src3_hbm.at[blk0 + ph], idxs_v)
            pltpu.sync_copy(dst3_hbm.at[blk0 + ph], idxd_v)

            def group(g, _):
                gds = []
                for b in range(NB_B):
                    j = g * NB_B + b
                    gds.append(pltpu.async_copy(
                        table_hbm.at[idxs_v.at[j]], rows_v.at[b], sem_g))
                scs = []
                for b in range(NB_B):
                    j = g * NB_B + b
                    gds[b].wait()
                    scs.append(pltpu.async_copy(
                        rows_v.at[b], acc.at[idxd_v.at[j]], sem_sc, add=True))
                for d in scs:
                    d.wait()
                return _

            lax.fori_loop(0, CPP_B // NB_B, group, None)
            for t in range((CPP_B // NB_B) * NB_B, CPP_B):
                pltpu.sync_copy(table_hbm.at[idxs_v.at[t]], rows_v.at[0])
                pltpu.sync_copy(rows_v.at[0], acc.at[idxd_v.at[t]], add=True)
            return _
        return phase

    @pl.when(c == 0)
    def _():
        lax.fori_loop(0, (EB // C) // CPP_B, make_phase(x_hbm), None)

    @pl.when(c == 1)
    def _():
        lax.fori_loop(0, (EB // C) // CPP_B, make_phase(agg_hbm), None)

    plsc.subcore_barrier()
    ds_ = []
    for j in range(RPT // C):
        r = s * RPT + j * C
        b = j % NB_B
        pltpu.sync_copy(acc.at[pl.ds(r, C)], rows_v.at[b])
        ds_.append(pltpu.async_copy(
            rows_v.at[b], neigh_hbm.at[c, pl.ds(r, C)], sem_sc))
        if b == NB_B - 1:
            for d in ds_:
                d.wait()
            ds_ = []
    for d in ds_:
        d.wait()


_RB = 1000


def _tc_combine_body(psum_ref, pdeg_ref, we_ref, be_ref, out_ref):
    ssum = psum_ref[0] + psum_ref[1]
    deg = jnp.sum(pdeg_ref[...], axis=0)[:, None]
    out_ref[...] = (
        jnp.dot(ssum, we_ref[...], preferred_element_type=jnp.float32)
        + deg * be_ref[...]
    )


def _tc_mlp_body(x_ref, agg_ref, n0_ref, n1_ref, w1a_ref, w1b_ref, b1_ref,
                 w2_ref, b2_ref, out_ref):
    pre0 = x_ref[...] + n0_ref[...]
    pre1 = agg_ref[...] + n1_ref[...]
    h = jax.nn.relu(
        jnp.dot(pre0, w1a_ref[...], preferred_element_type=jnp.float32)
        + jnp.dot(pre1, w1b_ref[...], preferred_element_type=jnp.float32)
        + b1_ref[...]
    )
    out_ref[...] = (
        jnp.dot(h, w2_ref[...], preferred_element_type=jnp.float32)
        + b2_ref[...]
    )


def kernel(x, edge_index, edge_attr, We, be, W1, b1, W2, b2):
    srcB = edge_index[0].reshape(E // (CPP_B * C), CPP_B, C)
    dstA = edge_index[1].reshape(E // (CPP_A * C), CPP_A, C)
    dstB = edge_index[1].reshape(E // (CPP_B * C), CPP_B, C)
    z128 = jnp.zeros((C, D), jnp.float32)
    zdeg = jnp.zeros((N_PAD,), jnp.float32)

    psum, pdeg = _sc_edge_agg(edge_attr, dstA, z128, zdeg)

    _CB = 1024
    agg = pl.pallas_call(
        _tc_combine_body,
        grid=(N_PAD // _CB,),
        in_specs=[
            pl.BlockSpec((NC, _CB, D), lambda i: (0, i, 0)),
            pl.BlockSpec((NC * NS, _CB), lambda i: (0, i)),
            pl.BlockSpec((D, D), lambda i: (0, 0)),
            pl.BlockSpec((1, D), lambda i: (0, 0)),
        ],
        out_specs=pl.BlockSpec((_CB, D), lambda i: (i, 0)),
        out_shape=jax.ShapeDtypeStruct((N_PAD, D), jnp.float32),
    )(psum, pdeg, We, be.reshape(1, D))

    neigh = _sc_neigh(x, agg, srcB, dstB, z128)

    out = pl.pallas_call(
        _tc_mlp_body,
        grid=(N // _RB,),
        in_specs=[
            pl.BlockSpec((_RB, D), lambda i: (i, 0)),
            pl.BlockSpec((_RB, D), lambda i: (i, 0)),
            pl.BlockSpec((_RB, D), lambda i: (i, 0)),
            pl.BlockSpec((_RB, D), lambda i: (i, 0)),
            pl.BlockSpec((D, D), lambda i: (0, 0)),
            pl.BlockSpec((D, D), lambda i: (0, 0)),
            pl.BlockSpec((1, D), lambda i: (0, 0)),
            pl.BlockSpec((D, D), lambda i: (0, 0)),
            pl.BlockSpec((1, D), lambda i: (0, 0)),
        ],
        out_specs=pl.BlockSpec((_RB, D), lambda i: (i, 0)),
        out_shape=jax.ShapeDtypeStruct((N, D), jnp.float32),
    )(x, agg, neigh[0], neigh[1], W1[:D], W1[D:], b1.reshape(1, D),
      W2, b2.reshape(1, D))

    return out
